# baseline (speedup 1.0000x reference)
import jax
import jax.numpy as jnp
from jax.experimental import pallas as pl
from jax.experimental.pallas import tpu as pltpu

_BM = 400


def _gcn_kernel(x_ref, w_ref, adj_ref, b_ref, out_ref, support_ref):
    i = pl.program_id(0)

    @pl.when(i == 0)
    def _():
        support_ref[...] = jnp.dot(
            x_ref[...], w_ref[...], preferred_element_type=jnp.float32
        )

    acc = jnp.dot(
        adj_ref[...], support_ref[...], preferred_element_type=jnp.float32
    )
    z = acc + b_ref[...]
    out_ref[...] = jnp.where(z > 0, z, jnp.exp(z) - 1.0)


def kernel(inputs, adj, weight, bias):
    n, in_f = inputs.shape
    out_f = weight.shape[1]
    bm = _BM
    bias2 = bias.reshape(1, out_f)
    return pl.pallas_call(
        _gcn_kernel,
        grid=(n // bm,),
        in_specs=[
            pl.BlockSpec((n, in_f), lambda i: (0, 0)),
            pl.BlockSpec((in_f, out_f), lambda i: (0, 0)),
            pl.BlockSpec((bm, n), lambda i: (i, 0)),
            pl.BlockSpec((1, out_f), lambda i: (0, 0)),
        ],
        out_specs=pl.BlockSpec((bm, out_f), lambda i: (i, 0)),
        out_shape=jax.ShapeDtypeStruct((n, out_f), jnp.float32),
        scratch_shapes=[pltpu.VMEM((n, out_f), jnp.float32)],
    )(inputs, weight, adj, bias2)


# speedup vs baseline: 1.0029x; 1.0029x over previous
import jax
import jax.numpy as jnp
from jax.experimental import pallas as pl
from jax.experimental.pallas import tpu as pltpu

_BM = 200


def _gcn_kernel(x_ref, w_ref, adja_ref, adjb_ref, b_ref, out_ref, support_ref):
    i = pl.program_id(0)

    @pl.when(i == 0)
    def _():
        support_ref[...] = jnp.dot(
            x_ref[...], w_ref[...], preferred_element_type=jnp.float32
        )

    s = support_ref[...]
    b = b_ref[...]
    za = jnp.dot(adja_ref[...], s, preferred_element_type=jnp.float32) + b
    out_ref[0] = jnp.where(za > 0, za, jnp.exp(za) - 1.0)
    zb = jnp.dot(adjb_ref[...], s, preferred_element_type=jnp.float32) + b
    out_ref[1] = jnp.where(zb > 0, zb, jnp.exp(zb) - 1.0)


def kernel(inputs, adj, weight, bias):
    n, in_f = inputs.shape
    out_f = weight.shape[1]
    bm = _BM
    nsteps = n // (2 * bm)
    bias2 = bias.reshape(1, out_f)
    out = pl.pallas_call(
        _gcn_kernel,
        grid=(nsteps,),
        in_specs=[
            pl.BlockSpec((n, in_f), lambda i: (0, 0)),
            pl.BlockSpec((in_f, out_f), lambda i: (0, 0)),
            pl.BlockSpec((bm, n), lambda i: (i, 0)),
            pl.BlockSpec((bm, n), lambda i: (i + nsteps, 0)),
            pl.BlockSpec((1, out_f), lambda i: (0, 0)),
        ],
        out_specs=pl.BlockSpec((2, bm, out_f), lambda i: (0, i, 0)),
        out_shape=jax.ShapeDtypeStruct((2, n // 2, out_f), jnp.float32),
        scratch_shapes=[pltpu.VMEM((n, out_f), jnp.float32)],
    )(inputs, weight, adj, adj, bias2)
    return out.reshape(n, out_f)
